# baseline (speedup 1.0000x reference)
import functools

import jax
import jax.numpy as jnp
from jax import lax
from jax.experimental import pallas as pl
from jax.experimental.pallas import tpu as pltpu
from jax.experimental.pallas import tpu_sc as plsc

N_NUM = 13
N_CAT = 26
N_EMB = 4
B = 4096
D = 128
VOCAB = 100000
EMB_DIM = 768
N_ALL = N_CAT + N_NUM + N_EMB

NW = 32
ROWS = B * N_CAT
RPW = ROWS // NW
CHUNK = 128
NCHUNK = RPW // CHUNK
NBUF = 4


def _sc_gather(tables, idx_grp):
    mesh = plsc.VectorSubcoreMesh(core_axis_name="c", subcore_axis_name="s")
    nblk = B // CHUNK

    @functools.partial(
        pl.kernel,
        out_type=jax.ShapeDtypeStruct((N_CAT, B, D), jnp.float32),
        mesh=mesh,
        scratch_types=[
            pltpu.VMEM((NCHUNK, CHUNK), jnp.int32),
            pltpu.VMEM((NBUF * CHUNK, D), jnp.float32),
            pltpu.SemaphoreType.DMA((NBUF,)),
            pltpu.SemaphoreType.DMA((NBUF,)),
        ],
    )
    def k(table_hbm, idx_hbm, out_hbm, idx_v, buf, gsem, ssem):
        wid = lax.axis_index("s") * 2 + lax.axis_index("c")
        pltpu.sync_copy(idx_hbm.at[wid], idx_v)

        def unit(c):
            u = wid * NCHUNK + c
            return u // nblk, (u % nblk) * CHUNK

        def bslice(s):
            return buf.at[pl.ds(s * CHUNK, CHUNK)]

        def start_gather(c, s):
            f, _ = unit(c)
            pltpu.async_copy(table_hbm.at[f].at[idx_v.at[c]], bslice(s),
                             gsem.at[s])

        def wait_gather(c, s):
            f, _ = unit(c)
            pltpu.make_async_copy(
                table_hbm.at[f].at[idx_v.at[c]], bslice(s), gsem.at[s]).wait()

        def out_slab(c):
            f, b0 = unit(c)
            return out_hbm.at[f].at[pl.ds(b0, CHUNK)]

        def start_store(c, s):
            pltpu.async_copy(bslice(s), out_slab(c), ssem.at[s])

        def wait_store(c, s):
            pltpu.make_async_copy(bslice(s), out_slab(c), ssem.at[s]).wait()

        start_gather(0, 0)
        start_gather(1, 1)
        start_gather(2, 2)

        def body(c, carry):
            s = lax.rem(c, NBUF)
            s2 = lax.rem(c + 3, NBUF)

            @pl.when(c >= 1)
            def _free_next_buf():
                wait_store(c - 1, s2)

            @pl.when(c + 3 < NCHUNK)
            def _launch_next_gather():
                start_gather(c + 3, s2)

            wait_gather(c, s)
            start_store(c, s)
            return carry

        lax.fori_loop(0, NCHUNK, body, 0)
        wait_store(NCHUNK - 1, lax.rem(NCHUNK - 1, NBUF))

    return k(tables, idx_grp)


def _ln(x, g, b):
    mu = jnp.mean(x, axis=-1, keepdims=True)
    xc = x - mu
    var = jnp.mean(xc * xc, axis=-1, keepdims=True)
    return xc * lax.rsqrt(var + 1e-5) * g + b


BBLK = 512
GRID = B // BBLK


def _tc_body(cat_ref, nf_ref, nw_ref, emb_ref, ew_ref, g_ref, be_ref, out_ref):
    g3 = g_ref[...].reshape(1, 1, D)
    be3 = be_ref[...].reshape(1, 1, D)
    out_ref[0:N_CAT] = _ln(cat_ref[...], g3, be3)
    nf = nf_ref[...]
    nw = nw_ref[...]
    numb = nf[:, :, None] * nw[:, None, :]
    out_ref[N_CAT:N_CAT + N_NUM] = _ln(numb, g3, be3)
    for n in range(N_EMB):
        e = jnp.dot(emb_ref[n], ew_ref[n], preferred_element_type=jnp.float32)
        out_ref[N_CAT + N_NUM + n] = _ln(e, g_ref[...], be_ref[...])


def _tc_fuse(cat_raw, nf, nw, emb, ew, g2, be2):
    return pl.pallas_call(
        _tc_body,
        grid=(GRID,),
        in_specs=[
            pl.BlockSpec((N_CAT, BBLK, D), lambda i: (0, i, 0)),
            pl.BlockSpec((N_NUM, BBLK), lambda i: (0, i)),
            pl.BlockSpec((N_NUM, D), lambda i: (0, 0)),
            pl.BlockSpec((N_EMB, BBLK, EMB_DIM), lambda i: (0, i, 0)),
            pl.BlockSpec((N_EMB, EMB_DIM, D), lambda i: (0, 0, 0)),
            pl.BlockSpec((1, D), lambda i: (0, 0)),
            pl.BlockSpec((1, D), lambda i: (0, 0)),
        ],
        out_specs=pl.BlockSpec((N_ALL, BBLK, D), lambda i: (0, i, 0)),
        out_shape=jax.ShapeDtypeStruct((N_ALL, B, D), jnp.float32),
    )(cat_raw, nf, nw, emb, ew, g2, be2)


def kernel(num_features, cat_features, emb_features, cat_tables, num_w, emb_w, ln_gamma, ln_beta):
    idx_grp = cat_features.reshape(NW, NCHUNK, CHUNK)
    cat_raw = _sc_gather(cat_tables, idx_grp)

    nf = num_features.reshape(N_NUM, B)
    nw = num_w.reshape(N_NUM, D)
    g2 = ln_gamma.reshape(1, D)
    be2 = ln_beta.reshape(1, D)
    out_fmaj = _tc_fuse(cat_raw, nf, nw, emb_features, emb_w, g2, be2)
    return jnp.transpose(out_fmaj, (1, 0, 2))
